# Initial kernel scaffold; baseline (speedup 1.0000x reference)
import jax
import jax.numpy as jnp
from jax.experimental import pallas as pl


def kernel(x, table, W, b):
    raise NotImplementedError("write your pallas kernel here")


# speedup vs baseline: 2.2356x; 2.2356x over previous
import functools

import jax
import jax.numpy as jnp
from jax import lax
from jax.experimental import pallas as pl
from jax.experimental.pallas import tpu as pltpu
from jax.experimental.pallas import tpu_sc as plsc

VOCAB = 100000
EMBED = 64
NUM_ENTITIES = 50
B = 4096
L = 50

TOKENS = B * L
D_PAD = 56


_ROWS_BLK = 2000


def _proj_body(t_ref, w_ref, b_ref, o_ref):
    o_ref[...] = (
        jnp.dot(t_ref[...], w_ref[...], preferred_element_type=jnp.float32)
        + b_ref[...]
    )


def _project(table, Wp, bp):
    return pl.pallas_call(
        _proj_body,
        grid=(VOCAB // _ROWS_BLK,),
        in_specs=[
            pl.BlockSpec((_ROWS_BLK, EMBED), lambda i: (i, 0)),
            pl.BlockSpec((EMBED, D_PAD), lambda i: (0, 0)),
            pl.BlockSpec((1, D_PAD), lambda i: (0, 0)),
        ],
        out_specs=pl.BlockSpec((_ROWS_BLK, D_PAD), lambda i: (i, 0)),
        out_shape=jax.ShapeDtypeStruct((VOCAB, D_PAD), jnp.float32),
    )(table, Wp, bp)


_NC = 2
_NS = 16
_NW = _NC * _NS
_CHUNK = 128
_PER_W = TOKENS // _NW
_NCHUNK = _PER_W // _CHUNK


def _gather(P, idx3):
    mesh = plsc.VectorSubcoreMesh(core_axis_name="c", subcore_axis_name="s")

    @functools.partial(
        pl.kernel,
        mesh=mesh,
        out_type=jax.ShapeDtypeStruct((TOKENS, D_PAD), jnp.float32),
        scratch_types=[
            pltpu.VMEM((_NCHUNK, _CHUNK), jnp.int32),
            pltpu.VMEM((_CHUNK, D_PAD), jnp.float32),
            pltpu.SemaphoreType.DMA,
        ],
        compiler_params=pltpu.CompilerParams(use_tc_tiling_on_sc=False),
    )
    def k(p_hbm, idx_hbm, out_hbm, idx_v, rows_v, sem):
        wid = lax.axis_index("s") * _NC + lax.axis_index("c")
        base = wid * _PER_W
        pltpu.sync_copy(idx_hbm.at[wid], idx_v)

        def body(j, _):
            pltpu.async_copy(p_hbm.at[idx_v.at[j]], rows_v, sem).wait()
            pltpu.sync_copy(
                rows_v, out_hbm.at[pl.ds(base + j * _CHUNK, _CHUNK)]
            )
            return 0

        lax.fori_loop(0, _NCHUNK, body, 0)

    return k(P, idx3)


def kernel(x, table, W, b):
    Wp = jnp.pad(W, ((0, 0), (0, D_PAD - NUM_ENTITIES)))
    bp = jnp.pad(b, (0, D_PAD - NUM_ENTITIES)).reshape(1, D_PAD)
    P = _project(table, Wp, bp)
    idx3 = x.reshape(-1).astype(jnp.int32).reshape(_NW, _NCHUNK, _CHUNK)
    out = _gather(P, idx3)
    return out[:, :NUM_ENTITIES].reshape(B, L, NUM_ENTITIES)


# speedup vs baseline: 2.9036x; 1.2988x over previous
import functools

import jax
import jax.numpy as jnp
from jax import lax
from jax.experimental import pallas as pl
from jax.experimental.pallas import tpu as pltpu
from jax.experimental.pallas import tpu_sc as plsc

VOCAB = 100000
EMBED = 64
NUM_ENTITIES = 50
B = 4096
L = 50

TOKENS = B * L
D_PAD = 128


_ROWS_BLK = 2000


def _proj_body(t_ref, w_ref, b_ref, o_ref):
    o_ref[...] = (
        jnp.dot(t_ref[...], w_ref[...], preferred_element_type=jnp.float32)
        + b_ref[...]
    )


def _project(table, Wp, bp):
    return pl.pallas_call(
        _proj_body,
        grid=(VOCAB // _ROWS_BLK,),
        in_specs=[
            pl.BlockSpec((_ROWS_BLK, EMBED), lambda i: (i, 0)),
            pl.BlockSpec((EMBED, D_PAD), lambda i: (0, 0)),
            pl.BlockSpec((1, D_PAD), lambda i: (0, 0)),
        ],
        out_specs=pl.BlockSpec((_ROWS_BLK, D_PAD), lambda i: (i, 0)),
        out_shape=jax.ShapeDtypeStruct((VOCAB, D_PAD), jnp.float32),
    )(table, Wp, bp)


_NC = 2
_NS = 16
_NW = _NC * _NS
_CHUNK = 128
_PER_W = TOKENS // _NW
_NCHUNK = _PER_W // _CHUNK


def _gather(P, idx3):
    mesh = plsc.VectorSubcoreMesh(core_axis_name="c", subcore_axis_name="s")

    @functools.partial(
        pl.kernel,
        mesh=mesh,
        out_type=jax.ShapeDtypeStruct((TOKENS, D_PAD), jnp.float32),
        scratch_types=[
            pltpu.VMEM((_NCHUNK, _CHUNK), jnp.int32),
            pltpu.VMEM((_CHUNK, D_PAD), jnp.float32),
            pltpu.SemaphoreType.DMA,
        ],
    )
    def k(p_hbm, idx_hbm, out_hbm, idx_v, rows_v, sem):
        wid = lax.axis_index("s") * _NC + lax.axis_index("c")
        base = wid * _PER_W
        pltpu.sync_copy(idx_hbm.at[wid], idx_v)

        def body(j, _):
            pltpu.async_copy(p_hbm.at[idx_v.at[j]], rows_v, sem).wait()
            pltpu.sync_copy(
                rows_v, out_hbm.at[pl.ds(base + j * _CHUNK, _CHUNK)]
            )
            return 0

        lax.fori_loop(0, _NCHUNK, body, 0)

    return k(P, idx3)


def kernel(x, table, W, b):
    Wp = jnp.pad(W, ((0, 0), (0, D_PAD - NUM_ENTITIES)))
    bp = jnp.pad(b, (0, D_PAD - NUM_ENTITIES)).reshape(1, D_PAD)
    P = _project(table, Wp, bp)
    idx3 = x.reshape(-1).astype(jnp.int32).reshape(_NW, _NCHUNK, _CHUNK)
    out = _gather(P, idx3)
    return out.reshape(B, L, D_PAD)[:, :, :NUM_ENTITIES]


# speedup vs baseline: 3.2824x; 1.1305x over previous
import functools

import jax
import jax.numpy as jnp
from jax import lax
from jax.experimental import pallas as pl
from jax.experimental.pallas import tpu as pltpu
from jax.experimental.pallas import tpu_sc as plsc

VOCAB = 100000
EMBED = 64
NUM_ENTITIES = 50
B = 4096
L = 50

D_PAD = 128


_ROWS_BLK = 2000


def _proj_body(t_ref, w_ref, b_ref, o_ref):
    o_ref[...] = (
        jnp.dot(t_ref[...], w_ref[...], preferred_element_type=jnp.float32)
        + b_ref[...]
    )


def _project(table, Wp, bp):
    return pl.pallas_call(
        _proj_body,
        grid=(VOCAB // _ROWS_BLK,),
        in_specs=[
            pl.BlockSpec((_ROWS_BLK, EMBED), lambda i: (i, 0)),
            pl.BlockSpec((EMBED, D_PAD), lambda i: (0, 0)),
            pl.BlockSpec((1, D_PAD), lambda i: (0, 0)),
        ],
        out_specs=pl.BlockSpec((_ROWS_BLK, D_PAD), lambda i: (i, 0)),
        out_shape=jax.ShapeDtypeStruct((VOCAB, D_PAD), jnp.float32),
    )(table, Wp, bp)


_NC = 2
_NS = 16
_NW = _NC * _NS
_ROWS_W = B // _NW


def _gather(P, xi):
    mesh = plsc.VectorSubcoreMesh(core_axis_name="c", subcore_axis_name="s")

    @functools.partial(
        pl.kernel,
        mesh=mesh,
        out_type=jax.ShapeDtypeStruct((B, L, D_PAD), jnp.float32),
        scratch_types=[
            pltpu.VMEM((_ROWS_W, L), jnp.int32),
            pltpu.VMEM((L, D_PAD), jnp.float32),
            pltpu.SemaphoreType.DMA,
        ],
    )
    def k(p_hbm, x_hbm, out_hbm, idx_v, rows_v, sem):
        wid = lax.axis_index("s") * _NC + lax.axis_index("c")
        base = wid * _ROWS_W
        pltpu.sync_copy(x_hbm.at[pl.ds(base, _ROWS_W)], idx_v)

        def body(r, _):
            pltpu.async_copy(p_hbm.at[idx_v.at[r]], rows_v, sem).wait()
            pltpu.sync_copy(rows_v, out_hbm.at[base + r])
            return 0

        lax.fori_loop(0, _ROWS_W, body, 0)

    return k(P, xi)


def kernel(x, table, W, b):
    Wp = jnp.pad(W, ((0, 0), (0, D_PAD - NUM_ENTITIES)))
    bp = jnp.pad(b, (0, D_PAD - NUM_ENTITIES)).reshape(1, D_PAD)
    P = _project(table, Wp, bp)
    out = _gather(P, x.astype(jnp.int32))
    return out[:, :, :NUM_ENTITIES]


# speedup vs baseline: 4.5574x; 1.3884x over previous
import functools

import jax
import jax.numpy as jnp
from jax import lax
from jax.experimental import pallas as pl
from jax.experimental.pallas import tpu as pltpu
from jax.experimental.pallas import tpu_sc as plsc

VOCAB = 100000
EMBED = 64
NUM_ENTITIES = 50
B = 4096
L = 50

D_PAD = 128


_ROWS_BLK = 10000


def _proj_body(t_ref, w_ref, b_ref, o_ref):
    o_ref[...] = (
        jnp.dot(t_ref[...], w_ref[...], preferred_element_type=jnp.float32)
        + b_ref[...]
    )


def _project(table, Wp, bp):
    return pl.pallas_call(
        _proj_body,
        grid=(VOCAB // _ROWS_BLK,),
        in_specs=[
            pl.BlockSpec((_ROWS_BLK, EMBED), lambda i: (i, 0)),
            pl.BlockSpec((EMBED, D_PAD), lambda i: (0, 0)),
            pl.BlockSpec((1, D_PAD), lambda i: (0, 0)),
        ],
        out_specs=pl.BlockSpec((_ROWS_BLK, D_PAD), lambda i: (i, 0)),
        out_shape=jax.ShapeDtypeStruct((VOCAB, D_PAD), jnp.float32),
    )(table, Wp, bp)


_NC = 2
_NS = 16
_NW = _NC * _NS
_ROWS_W = B // _NW
_HALF = _ROWS_W // 2


def _gather(P, xp):
    mesh = plsc.VectorSubcoreMesh(core_axis_name="c", subcore_axis_name="s")

    @functools.partial(
        pl.kernel,
        mesh=mesh,
        out_type=jax.ShapeDtypeStruct((B, L, D_PAD), jnp.float32),
        scratch_types=[
            pltpu.VMEM((_ROWS_W, D_PAD), jnp.int32),
            pltpu.VMEM((L, D_PAD), jnp.float32),
            pltpu.VMEM((L, D_PAD), jnp.float32),
            pltpu.SemaphoreType.DMA,
            pltpu.SemaphoreType.DMA,
        ],
    )
    def k(p_hbm, x_hbm, out_hbm, idx_v, buf0, buf1, sem0, sem1):
        wid = lax.axis_index("s") * _NC + lax.axis_index("c")
        base = wid * _ROWS_W
        pltpu.sync_copy(x_hbm.at[pl.ds(base, _ROWS_W)], idx_v)

        def fetch(r, buf, sem):
            return pltpu.make_async_copy(
                p_hbm.at[idx_v.at[r, pl.ds(0, L)]], buf, sem
            )

        fetch(0, buf0, sem0).start()

        def body(i, _):
            r0 = 2 * i
            fetch(r0 + 1, buf1, sem1).start()
            fetch(r0, buf0, sem0).wait()
            pltpu.sync_copy(buf0, out_hbm.at[base + r0])

            @pl.when(i + 1 < _HALF)
            def _():
                fetch(r0 + 2, buf0, sem0).start()

            fetch(r0 + 1, buf1, sem1).wait()
            pltpu.sync_copy(buf1, out_hbm.at[base + r0 + 1])
            return 0

        lax.fori_loop(0, _HALF, body, 0)

    return k(P, xp)


def kernel(x, table, W, b):
    Wp = jnp.pad(W, ((0, 0), (0, D_PAD - NUM_ENTITIES)))
    bp = jnp.pad(b, (0, D_PAD - NUM_ENTITIES)).reshape(1, D_PAD)
    P = _project(table, Wp, bp)
    xp = jnp.pad(x.astype(jnp.int32), ((0, 0), (0, D_PAD - L)))
    out = _gather(P, xp)
    return out[:, :, :NUM_ENTITIES]


# speedup vs baseline: 5.1199x; 1.1234x over previous
import functools

import jax
import jax.numpy as jnp
from jax import lax
from jax.experimental import pallas as pl
from jax.experimental.pallas import tpu as pltpu
from jax.experimental.pallas import tpu_sc as plsc

VOCAB = 100000
EMBED = 64
NUM_ENTITIES = 50
B = 4096
L = 50

D_PAD = 128


_ROWS_BLK = 10000


def _proj_body(t_ref, w_ref, b_ref, o_ref):
    o_ref[...] = (
        jnp.dot(t_ref[...], w_ref[...], preferred_element_type=jnp.float32)
        + b_ref[...]
    )


def _project(table, Wp, bp):
    return pl.pallas_call(
        _proj_body,
        grid=(VOCAB // _ROWS_BLK,),
        in_specs=[
            pl.BlockSpec((_ROWS_BLK, EMBED), lambda i: (i, 0)),
            pl.BlockSpec((EMBED, D_PAD), lambda i: (0, 0)),
            pl.BlockSpec((1, D_PAD), lambda i: (0, 0)),
        ],
        out_specs=pl.BlockSpec((_ROWS_BLK, D_PAD), lambda i: (i, 0)),
        out_shape=jax.ShapeDtypeStruct((VOCAB, D_PAD), jnp.float32),
    )(table, Wp, bp)


_NC = 2
_NS = 16
_NW = _NC * _NS
_ROWS_W = B // _NW
_DEPTH = 4
_NITER = _ROWS_W // _DEPTH


def _gather(P, xp):
    mesh = plsc.VectorSubcoreMesh(core_axis_name="c", subcore_axis_name="s")

    @functools.partial(
        pl.kernel,
        mesh=mesh,
        out_type=jax.ShapeDtypeStruct((B, L, D_PAD), jnp.float32),
        scratch_types=[
            pltpu.VMEM((_ROWS_W, D_PAD), jnp.int32),
            [pltpu.VMEM((L, D_PAD), jnp.float32) for _ in range(_DEPTH)],
            [pltpu.SemaphoreType.DMA for _ in range(_DEPTH)],
        ],
    )
    def k(p_hbm, x_hbm, out_hbm, idx_v, bufs, sems):
        wid = lax.axis_index("s") * _NC + lax.axis_index("c")
        base = wid * _ROWS_W
        pltpu.sync_copy(x_hbm.at[pl.ds(base, _ROWS_W)], idx_v)

        def fetch(r, b):
            return pltpu.make_async_copy(
                p_hbm.at[idx_v.at[r, pl.ds(0, L)]], bufs[b], sems[b]
            )

        for b in range(_DEPTH - 1):
            fetch(b, b).start()

        def body(i, _):
            r0 = i * _DEPTH
            for b in range(_DEPTH):
                nxt = r0 + b + _DEPTH - 1
                tgt = (b - 1) % _DEPTH

                @pl.when(nxt < _ROWS_W)
                def _(nxt=nxt, tgt=tgt):
                    fetch(nxt, tgt).start()

                fetch(r0 + b, b).wait()
                pltpu.sync_copy(bufs[b], out_hbm.at[base + r0 + b])
            return 0

        lax.fori_loop(0, _NITER, body, 0)

    return k(P, xp)


def kernel(x, table, W, b):
    Wp = jnp.pad(W, ((0, 0), (0, D_PAD - NUM_ENTITIES)))
    bp = jnp.pad(b, (0, D_PAD - NUM_ENTITIES)).reshape(1, D_PAD)
    P = _project(table, Wp, bp)
    xp = jnp.pad(x.astype(jnp.int32), ((0, 0), (0, D_PAD - L)))
    out = _gather(P, xp)
    return out[:, :, :NUM_ENTITIES]


# speedup vs baseline: 5.1347x; 1.0029x over previous
import functools

import jax
import jax.numpy as jnp
from jax import lax
from jax.experimental import pallas as pl
from jax.experimental.pallas import tpu as pltpu
from jax.experimental.pallas import tpu_sc as plsc

VOCAB = 100000
EMBED = 64
NUM_ENTITIES = 50
B = 4096
L = 50

D_PAD = 128


_ROWS_BLK = 10000


def _proj_body(t_ref, w_ref, b_ref, o_ref):
    o_ref[...] = (
        jnp.dot(t_ref[...], w_ref[...], preferred_element_type=jnp.float32)
        + b_ref[...]
    )


def _project(table, Wp, bp):
    return pl.pallas_call(
        _proj_body,
        grid=(VOCAB // _ROWS_BLK,),
        in_specs=[
            pl.BlockSpec((_ROWS_BLK, EMBED), lambda i: (i, 0)),
            pl.BlockSpec((EMBED, D_PAD), lambda i: (0, 0)),
            pl.BlockSpec((1, D_PAD), lambda i: (0, 0)),
        ],
        out_specs=pl.BlockSpec((_ROWS_BLK, D_PAD), lambda i: (i, 0)),
        out_shape=jax.ShapeDtypeStruct((VOCAB, D_PAD), jnp.float32),
    )(table, Wp, bp)


_NC = 2
_NS = 16
_NW = _NC * _NS
_ROWS_W = B // _NW
_DEPTH = 8
_NITER = _ROWS_W // _DEPTH


def _gather(P, xp):
    mesh = plsc.VectorSubcoreMesh(core_axis_name="c", subcore_axis_name="s")

    @functools.partial(
        pl.kernel,
        mesh=mesh,
        out_type=jax.ShapeDtypeStruct((B, L, D_PAD), jnp.float32),
        scratch_types=[
            pltpu.VMEM((_ROWS_W, D_PAD), jnp.int32),
            [pltpu.VMEM((L, D_PAD), jnp.float32) for _ in range(_DEPTH)],
            [pltpu.SemaphoreType.DMA for _ in range(_DEPTH)],
        ],
    )
    def k(p_hbm, x_hbm, out_hbm, idx_v, bufs, sems):
        wid = lax.axis_index("s") * _NC + lax.axis_index("c")
        base = wid * _ROWS_W
        pltpu.sync_copy(x_hbm.at[pl.ds(base, _ROWS_W)], idx_v)

        def fetch(r, b):
            return pltpu.make_async_copy(
                p_hbm.at[idx_v.at[r, pl.ds(0, L)]], bufs[b], sems[b]
            )

        for b in range(_DEPTH - 1):
            fetch(b, b).start()

        def body(i, _):
            r0 = i * _DEPTH
            for b in range(_DEPTH):
                nxt = r0 + b + _DEPTH - 1
                tgt = (b - 1) % _DEPTH

                @pl.when(nxt < _ROWS_W)
                def _(nxt=nxt, tgt=tgt):
                    fetch(nxt, tgt).start()

                fetch(r0 + b, b).wait()
                pltpu.sync_copy(bufs[b], out_hbm.at[base + r0 + b])
            return 0

        lax.fori_loop(0, _NITER, body, 0)

    return k(P, xp)


def kernel(x, table, W, b):
    Wp = jnp.pad(W, ((0, 0), (0, D_PAD - NUM_ENTITIES)))
    bp = jnp.pad(b, (0, D_PAD - NUM_ENTITIES)).reshape(1, D_PAD)
    P = _project(table, Wp, bp)
    xp = jnp.pad(x.astype(jnp.int32), ((0, 0), (0, D_PAD - L)))
    out = _gather(P, xp)
    return out[:, :, :NUM_ENTITIES]


# speedup vs baseline: 6.3617x; 1.2390x over previous
import functools

import jax
import jax.numpy as jnp
from jax import lax
from jax.experimental import pallas as pl
from jax.experimental.pallas import tpu as pltpu
from jax.experimental.pallas import tpu_sc as plsc

VOCAB = 100000
EMBED = 64
NUM_ENTITIES = 50
B = 4096
L = 50

D_PAD = 128


_ROWS_BLK = 12800


def _proj_body(t_ref, w_ref, b_ref, o_ref):
    o_ref[...] = (
        jax.lax.dot_general(
            t_ref[...], w_ref[...],
            (((0,), (0,)), ((), ())),
            preferred_element_type=jnp.float32,
        )
        + b_ref[...]
    )


def _project(tableT, Wp, bp):
    return pl.pallas_call(
        _proj_body,
        grid=((VOCAB + _ROWS_BLK - 1) // _ROWS_BLK,),
        in_specs=[
            pl.BlockSpec((EMBED, _ROWS_BLK), lambda i: (0, i)),
            pl.BlockSpec((EMBED, D_PAD), lambda i: (0, 0)),
            pl.BlockSpec((1, D_PAD), lambda i: (0, 0)),
        ],
        out_specs=pl.BlockSpec((_ROWS_BLK, D_PAD), lambda i: (i, 0)),
        out_shape=jax.ShapeDtypeStruct((VOCAB, D_PAD), jnp.float32),
    )(tableT, Wp, bp)


_NC = 2
_NS = 16
_NW = _NC * _NS
_ROWS_W = B // _NW
_DEPTH = 8
_NITER = _ROWS_W // _DEPTH


def _gather(P, xp):
    mesh = plsc.VectorSubcoreMesh(core_axis_name="c", subcore_axis_name="s")

    @functools.partial(
        pl.kernel,
        mesh=mesh,
        out_type=jax.ShapeDtypeStruct((B, L, D_PAD), jnp.float32),
        scratch_types=[
            pltpu.VMEM((_ROWS_W, D_PAD), jnp.int32),
            [pltpu.VMEM((L, D_PAD), jnp.float32) for _ in range(_DEPTH)],
            [pltpu.SemaphoreType.DMA for _ in range(_DEPTH)],
        ],
    )
    def k(p_hbm, x_hbm, out_hbm, idx_v, bufs, sems):
        wid = lax.axis_index("s") * _NC + lax.axis_index("c")
        base = wid * _ROWS_W
        pltpu.sync_copy(x_hbm.at[pl.ds(base, _ROWS_W)], idx_v)

        def fetch(r, b):
            return pltpu.make_async_copy(
                p_hbm.at[idx_v.at[r, pl.ds(0, L)]], bufs[b], sems[b]
            )

        for b in range(_DEPTH - 1):
            fetch(b, b).start()

        def body(i, _):
            r0 = i * _DEPTH
            for b in range(_DEPTH):
                nxt = r0 + b + _DEPTH - 1
                tgt = (b - 1) % _DEPTH

                @pl.when(nxt < _ROWS_W)
                def _(nxt=nxt, tgt=tgt):
                    fetch(nxt, tgt).start()

                fetch(r0 + b, b).wait()
                pltpu.sync_copy(bufs[b], out_hbm.at[base + r0 + b])
            return 0

        lax.fori_loop(0, _NITER, body, 0)

    return k(P, xp)


def kernel(x, table, W, b):
    Wp = jnp.pad(W, ((0, 0), (0, D_PAD - NUM_ENTITIES)))
    bp = jnp.pad(b, (0, D_PAD - NUM_ENTITIES)).reshape(1, D_PAD)
    P = _project(table.T, Wp, bp)
    xp = jnp.pad(x.astype(jnp.int32), ((0, 0), (0, D_PAD - L)))
    out = _gather(P, xp)
    return out[:, :, :NUM_ENTITIES]
